# baseline (speedup 1.0000x reference)
import functools

import jax
import jax.numpy as jnp
from jax import lax
from jax.experimental import pallas as pl
from jax.experimental.pallas import tpu as pltpu
from jax.experimental.pallas import tpu_sc as plsc

N = 10000
T = 8
H = 128
MAXOBJ = 20000

NC = 2
NS = 16
NW = NC * NS

B = 80
CR = 80000 // B
C2 = 10000 // B
ZB = B
CZ = N // ZB
KZ = -(-CZ // NS)


def _make_steps(counts):
  steps = [(s, t) for s, cnt in enumerate(counts) for t in range(cnt // NW)]
  tails = [(s, cnt // NW, cnt) for s, cnt in enumerate(counts) if cnt % NW]
  return steps, tails


def _mesh():
  return plsc.VectorSubcoreMesh(core_axis_name="c", subcore_axis_name="s",
                                num_cores=NC, num_subcores=NS)


def _gather_pipeline(ns_hbm, vs, xs, idxb, rowb, si, sg, ss, wid, steps,
                     tails):
  def src_dst(j):
    s, t = steps[j]
    return vs[s], xs[s], wid + NW * t

  def iload(j):
    v, _, c = src_dst(j)
    return pltpu.async_copy(v.at[pl.ds(c * B, B)], idxb[j % 3], si[j % 3])

  def gath(j):
    m = j % 3
    return pltpu.async_copy(ns_hbm.at[idxb[m]], rowb[m], sg[m])

  def store(j):
    _, x, c = src_dst(j)
    m = j % 3
    return pltpu.async_copy(rowb[m], x.at[pl.ds(c * B, B)], ss[m])

  NK = len(steps)
  d = {}
  d["i", 0] = iload(0)
  d["i", 1] = iload(1)
  for j in range(NK):
    d["i", j].wait()
    if j >= 3:
      d["s", j - 3].wait()
    d["g", j] = gath(j)
    if j >= 1:
      d["g", j - 1].wait()
      d["s", j - 1] = store(j - 1)
    if j + 2 < NK:
      d["i", j + 2] = iload(j + 2)
  d["g", NK - 1].wait()
  d["s", NK - 1] = store(NK - 1)
  d["s", NK - 3].wait()
  d["s", NK - 2].wait()
  d["s", NK - 1].wait()

  for s, t, cmax in tails:
    v, x = vs[s], xs[s]
    c = wid + NW * t

    @pl.when(c < cmax)
    def _(v=v, x=x, c=c):
      pltpu.sync_copy(v.at[pl.ds(c * B, B)], idxb[0])
      pltpu.async_copy(ns_hbm.at[idxb[0]], rowb[0], si[0]).wait()
      pltpu.sync_copy(rowb[0], x.at[pl.ds(c * B, B)])


_GATHER_SCRATCH = ([pltpu.VMEM((B,), jnp.int32)] * 3
                   + [pltpu.VMEM((B, H), jnp.float32)] * 3
                   + [pltpu.SemaphoreType.DMA] * 9)


@functools.cache
def _build_gather2():
  steps, tails = _make_steps((CR, CR))

  @functools.partial(
      pl.kernel,
      out_type=(jax.ShapeDtypeStruct((CR * B, H), jnp.float32),
                jax.ShapeDtypeStruct((CR * B, H), jnp.float32)),
      mesh=_mesh(), scratch_types=list(_GATHER_SCRATCH),
  )
  def k(ns_hbm, v0, v1, x0, x1, ib0, ib1, ib2, rb0, rb1, rb2, *sems):
    wid = lax.axis_index("s") * NC + lax.axis_index("c")
    _gather_pipeline(ns_hbm, (v0, v1), (x0, x1), (ib0, ib1, ib2),
                     (rb0, rb1, rb2), sems[0:3], sems[3:6], sems[6:9],
                     wid, steps, tails)

  return k


@functools.cache
def _build_gather3():
  steps, tails = _make_steps((CR, CR, C2))

  @functools.partial(
      pl.kernel,
      out_type=(jax.ShapeDtypeStruct((CR * B, H), jnp.float32),
                jax.ShapeDtypeStruct((CR * B, H), jnp.float32),
                jax.ShapeDtypeStruct((C2 * B, H), jnp.float32)),
      mesh=_mesh(), scratch_types=list(_GATHER_SCRATCH),
  )
  def k(ns_hbm, v0, v1, v2, x0, x1, x2,
        ib0, ib1, ib2, rb0, rb1, rb2, *sems):
    wid = lax.axis_index("s") * NC + lax.axis_index("c")
    _gather_pipeline(ns_hbm, (v0, v1, v2), (x0, x1, x2), (ib0, ib1, ib2),
                     (rb0, rb1, rb2), sems[0:3], sems[3:6], sems[6:9],
                     wid, steps, tails)

  return k


def _scatter_pipeline(es, vs, acc, out_hbm, idxb, bufb, si, sl, sa,
                      cid, sid, wid, steps, tails):
  def zrow(r, carry):
    for j in range(H // 16):
      bufb[0][r, pl.ds(j * 16, 16)] = jnp.zeros((16,), jnp.float32)
    return carry

  lax.fori_loop(0, ZB, zrow, 0)

  def zchunk(k, carry):
    c = sid + NS * k

    @pl.when(c < CZ)
    def _():
      pltpu.sync_copy(bufb[0], acc.at[pl.ds(c * ZB, ZB)])
    return carry

  lax.fori_loop(0, KZ, zchunk, 0)
  plsc.subcore_barrier()

  def srcs(j):
    s, t = steps[j]
    return es[s], vs[s], wid + NW * t

  def iload(j):
    _, v, c = srcs(j)
    m = j % 3
    return pltpu.async_copy(v.at[pl.ds(c * B, B)], idxb[m], si[m])

  def dload(j):
    e, _, c = srcs(j)
    m = j % 3
    return pltpu.async_copy(e.at[pl.ds(c * B, B)], bufb[m], sl[m])

  def addc(j):
    m = j % 3
    return pltpu.async_copy(bufb[m], acc.at[idxb[m]], sa[m], add=True)

  NK = len(steps)
  d = {}
  d["i", 0] = iload(0)
  d["l", 0] = dload(0)
  d["i", 1] = iload(1)
  d["l", 1] = dload(1)
  for j in range(NK):
    d["i", j].wait()
    d["l", j].wait()
    d["a", j] = addc(j)
    if j >= 1:
      d["a", j - 1].wait()
    if j + 2 < NK:
      d["i", j + 2] = iload(j + 2)
      d["l", j + 2] = dload(j + 2)
  d["a", NK - 1].wait()

  for s, t, cmax in tails:
    e, v = es[s], vs[s]
    c = wid + NW * t

    @pl.when(c < cmax)
    def _(e=e, v=v, c=c):
      pltpu.sync_copy(v.at[pl.ds(c * B, B)], idxb[0])
      pltpu.sync_copy(e.at[pl.ds(c * B, B)], bufb[0])
      pltpu.sync_copy(bufb[0], acc.at[idxb[0]], add=True)

  plsc.subcore_barrier()

  def ochunk(k, carry):
    c = sid + NS * k

    @pl.when(c < CZ)
    def _():
      pltpu.sync_copy(acc.at[pl.ds(c * ZB, ZB)],
                      out_hbm.at[cid, pl.ds(c * ZB, ZB)])
    return carry

  lax.fori_loop(0, KZ, ochunk, 0)


_SCATTER_SCRATCH = ([pltpu.VMEM((B,), jnp.int32)] * 3
                    + [pltpu.VMEM((B, H), jnp.float32)] * 3
                    + [pltpu.VMEM_SHARED((N, H), jnp.float32)]
                    + [pltpu.SemaphoreType.DMA] * 9)


@functools.cache
def _build_scatter5():
  steps, tails = _make_steps((CR, CR, CR, CR, C2))

  @functools.partial(
      pl.kernel,
      out_type=jax.ShapeDtypeStruct((NC, N, H), jnp.float32),
      mesh=_mesh(), scratch_types=list(_SCATTER_SCRATCH),
  )
  def k(e0, e1, e2, e3, e4, v0, v1, v2, v3, v4, out_hbm,
        ib0, ib1, ib2, bb0, bb1, bb2, acc, *sems):
    cid = lax.axis_index("c")
    sid = lax.axis_index("s")
    wid = sid * NC + cid
    _scatter_pipeline((e0, e1, e2, e3, e4), (v0, v1, v2, v3, v4), acc,
                      out_hbm, (ib0, ib1, ib2), (bb0, bb1, bb2),
                      sems[0:3], sems[3:6], sems[6:9],
                      cid, sid, wid, steps, tails)

  return k


def _prep_body(has_extra, ns_ref, wgeT_ref, wex0_ref, wexCT_ref, b1_ref,
               c0_ref):
    x = ns_ref[...]
    off = jnp.max(x, axis=0, keepdims=True)
    s = jnp.sum(jnp.exp((x - off) * 8.0), axis=0, keepdims=True)
    ge = 0.125 * jnp.log(s) + off
    c0 = jnp.dot(ge, wgeT_ref[...], preferred_element_type=jnp.float32)
    c0 = c0 + b1_ref[...]
    if has_extra:
        counts = jnp.sum(x[:, :T], axis=0, keepdims=True)
        c0 = c0 + (N / float(MAXOBJ)) * wex0_ref[...]
        c0 = c0 + jnp.dot(counts * (1.0 / N), wexCT_ref[...],
                          preferred_element_type=jnp.float32)
    c0_ref[...] = c0


def _prep(ns, wgeT, wex0, wexCT, b1, has_extra):
    return pl.pallas_call(
        functools.partial(_prep_body, has_extra),
        out_shape=jax.ShapeDtypeStruct((1, 2 * H), jnp.float32),
    )(ns, wgeT, wex0, wexCT, b1)


def _mlpR_body(xa_ref, xb_ref, w1T_ref, b1_ref, w2T_ref, b2_ref,
               ea_ref, eb_ref):
    x16 = jnp.concatenate([xa_ref[...], xb_ref[...]],
                          axis=1).astype(jnp.bfloat16)
    h = jnp.maximum(
        jnp.dot(x16, w1T_ref[...], preferred_element_type=jnp.float32)
        + b1_ref[...], 0.0)
    o = jnp.dot(h.astype(jnp.bfloat16), w2T_ref[...],
                preferred_element_type=jnp.float32) + b2_ref[...]
    e = jnp.exp(8.0 * o)
    ea_ref[...] = e[:, :H]
    eb_ref[...] = e[:, H:]


def _mlpR(xa, xb, w1T, b1, w2T, b2, bf):
    nb = 80000 // bf
    d = 2 * H
    espec = jax.ShapeDtypeStruct((80000, H), jnp.float32)
    return pl.pallas_call(
        _mlpR_body,
        grid=(nb,),
        in_specs=[pl.BlockSpec((bf, H), lambda i: (i, 0)),
                  pl.BlockSpec((bf, H), lambda i: (i, 0)),
                  pl.BlockSpec((d, d), lambda i: (0, 0)),
                  pl.BlockSpec((1, d), lambda i: (0, 0)),
                  pl.BlockSpec((d, d), lambda i: (0, 0)),
                  pl.BlockSpec((1, d), lambda i: (0, 0))],
        out_specs=[pl.BlockSpec((bf, H), lambda i: (i, 0)),
                   pl.BlockSpec((bf, H), lambda i: (i, 0))],
        out_shape=(espec, espec),
    )(xa, xb, w1T, b1, w2T, b2)


def _mlp2_body(x_ref, w1T_ref, b1_ref, w2T_ref, b2_ref, o_ref):
    x16 = x_ref[...].astype(jnp.bfloat16)
    h = jnp.maximum(
        jnp.dot(x16, w1T_ref[...], preferred_element_type=jnp.float32)
        + b1_ref[...], 0.0)
    o = jnp.dot(h.astype(jnp.bfloat16), w2T_ref[...],
                preferred_element_type=jnp.float32)
    o_ref[...] = jnp.exp(8.0 * (o + b2_ref[...]))


def _mlp2(x2, w1T, b1, w2T, b2, bf):
    nb = 10000 // bf
    return pl.pallas_call(
        _mlp2_body,
        grid=(nb,),
        in_specs=[pl.BlockSpec((bf, H), lambda i: (i, 0)),
                  pl.BlockSpec((H, H), lambda i: (0, 0)),
                  pl.BlockSpec((1, H), lambda i: (0, 0)),
                  pl.BlockSpec((H, H), lambda i: (0, 0)),
                  pl.BlockSpec((1, H), lambda i: (0, 0))],
        out_specs=pl.BlockSpec((bf, H), lambda i: (i, 0)),
        out_shape=jax.ShapeDtypeStruct((10000, H), jnp.float32),
    )(x2, w1T, b1, w2T, b2)


def _upd_body(p_ref, ns_ref, c0_ref, w1mT_ref, w1nsT_ref, w2T_ref,
              b2_ref, o_ref):
    p = p_ref[0] + p_ref[1]
    m = 0.125 * jnp.log(p + 1e-16)
    h = jnp.maximum(
        jnp.dot(m, w1mT_ref[...], preferred_element_type=jnp.float32)
        + jnp.dot(ns_ref[...], w1nsT_ref[...],
                  preferred_element_type=jnp.float32)
        + c0_ref[...], 0.0)
    o_ref[...] = jnp.dot(h, w2T_ref[...],
                         preferred_element_type=jnp.float32) + b2_ref[...]


def _upd(p, ns, c0, w1mT, w1nsT, w2T, b2, bf):
    nb = N // bf
    return pl.pallas_call(
        _upd_body,
        grid=(nb,),
        in_specs=[pl.BlockSpec((NC, bf, H), lambda i: (0, i, 0)),
                  pl.BlockSpec((bf, H), lambda i: (i, 0)),
                  pl.BlockSpec((1, 2 * H), lambda i: (0, 0)),
                  pl.BlockSpec((H, 2 * H), lambda i: (0, 0)),
                  pl.BlockSpec((H, 2 * H), lambda i: (0, 0)),
                  pl.BlockSpec((2 * H, H), lambda i: (0, 0)),
                  pl.BlockSpec((1, H), lambda i: (0, 0))],
        out_specs=pl.BlockSpec((bf, H), lambda i: (i, 0)),
        out_shape=jax.ShapeDtypeStruct((N, H), jnp.float32),
    )(p, ns, c0, w1mT, w1nsT, w2T, b2)


def _gather2_fn(ns, v0, v1):
    return _build_gather2()(ns, v0, v1)


def _gather3_fn(ns, v0, v1, v2):
    return _build_gather3()(ns, v0, v1, v2)


def _scatter5_fn(e0, e1, e2, e3, e4, v0, v1, v2, v3, v4):
    return _build_scatter5()(e0, e1, e2, e3, e4, v0, v1, v2, v3, v4)


def kernel(type_ids, rel0_values, rel1_values, rel2_values, init_random,
           r0_W1, r0_b1, r0_W2, r0_b2, r1_W1, r1_b1, r1_W2, r1_b2,
           r2_W1, r2_b1, r2_W2, r2_b2,
           u_W1, u_b1, u_W2, u_b2, v_W1, v_b1, v_W2, v_b2):
    f32 = jnp.float32
    bf16 = jnp.bfloat16
    ns = jnp.concatenate(
        [jax.nn.one_hot(type_ids, T, dtype=f32), init_random], axis=1)
    r0v = rel0_values.astype(jnp.int32)
    r1v = rel1_values.astype(jnp.int32)
    va0, vb0 = r0v[0::2], r0v[1::2]
    va1, vb1 = r1v[0::2], r1v[1::2]
    v2 = rel2_values.astype(jnp.int32)

    w1T0 = r0_W1.T.astype(bf16)
    w2T0 = r0_W2.T.astype(bf16)
    b10 = r0_b1.reshape(1, 2 * H)
    b20 = r0_b2.reshape(1, 2 * H)
    w1T1 = r1_W1.T.astype(bf16)
    w2T1 = r1_W2.T.astype(bf16)
    b11 = r1_b1.reshape(1, 2 * H)
    b21 = r1_b2.reshape(1, 2 * H)
    w1T2 = r2_W1.T.astype(bf16)
    w2T2 = r2_W2.T.astype(bf16)
    b12 = r2_b1.reshape(1, H)
    b22 = r2_b2.reshape(1, H)

    E = T + 1
    v_geT = v_W1[:, E:E + H].T
    v_mT = v_W1[:, E + H:E + 2 * H].T
    v_nsT = v_W1[:, E + 2 * H:].T
    v_ex0 = v_W1[:, 0:1].T
    v_exCT = v_W1[:, 1:E].T
    u_geT = u_W1[:, :H].T
    u_mT = u_W1[:, H:2 * H].T
    u_nsT = u_W1[:, 2 * H:].T
    zpad = jnp.zeros((T, 2 * H), f32)
    zpad1 = jnp.zeros((1, 2 * H), f32)

    for it in range(2):
        if it == 0:
            c0 = _prep(ns, v_geT, v_ex0, v_exCT, v_b1.reshape(1, -1), True)
            w1mT, w1nsT = v_mT, v_nsT
            w2T, b2 = v_W2.T, v_b2.reshape(1, H)
        else:
            c0 = _prep(ns, u_geT, zpad1, zpad, u_b1.reshape(1, -1), False)
            w1mT, w1nsT = u_mT, u_nsT
            w2T, b2 = u_W2.T, u_b2.reshape(1, H)

        xa0, xb0 = _gather2_fn(ns, va0, vb0)
        xa1, xb1, x2 = _gather3_fn(ns, va1, vb1, v2)
        ea0, eb0 = _mlpR(xa0, xb0, w1T0, b10, w2T0, b20, 1000)
        ea1, eb1 = _mlpR(xa1, xb1, w1T1, b11, w2T1, b21, 1000)
        e2 = _mlp2(x2, w1T2, b12, w2T2, b22, 1000)
        p = _scatter5_fn(ea0, eb0, ea1, eb1, e2, va0, vb0, va1, vb1, v2)
        ns = _upd(p, ns, c0, w1mT, w1nsT, w2T, b2, 1000)
    return ns


# speedup vs baseline: 1.0893x; 1.0893x over previous
import functools

import jax
import jax.numpy as jnp
from jax import lax
from jax.experimental import pallas as pl
from jax.experimental.pallas import tpu as pltpu
from jax.experimental.pallas import tpu_sc as plsc

N = 10000
T = 8
H = 128
MAXOBJ = 20000

NC = 2
NS = 16
NW = NC * NS

B = 80
CR = 80000 // B
C2 = 10000 // B
ZB = B
CZ = N // ZB
KZ = -(-CZ // NS)


def _make_steps(counts):
  steps = [(s, t) for s, cnt in enumerate(counts) for t in range(cnt // NW)]
  tails = [(s, cnt // NW, cnt) for s, cnt in enumerate(counts) if cnt % NW]
  return steps, tails


def _mesh():
  return plsc.VectorSubcoreMesh(core_axis_name="c", subcore_axis_name="s",
                                num_cores=NC, num_subcores=NS)


def _gather_pipeline(ns_hbm, vs, xs, idxb, rowb, si, sg, ss, wid, steps,
                     tails):
  def src_dst(j):
    s, t = steps[j]
    return vs[s], xs[s], wid + NW * t

  def iload(j):
    v, _, c = src_dst(j)
    return pltpu.async_copy(v.at[pl.ds(c * B, B)], idxb[j % 3], si[j % 3])

  def gath(j):
    m = j % 3
    return pltpu.async_copy(ns_hbm.at[idxb[m]], rowb[m], sg[m])

  def store(j):
    _, x, c = src_dst(j)
    m = j % 3
    return pltpu.async_copy(rowb[m], x.at[pl.ds(c * B, B)], ss[m])

  NK = len(steps)
  d = {}
  d["i", 0] = iload(0)
  d["i", 1] = iload(1)
  for j in range(NK):
    d["i", j].wait()
    if j >= 3:
      d["s", j - 3].wait()
    d["g", j] = gath(j)
    if j >= 1:
      d["g", j - 1].wait()
      d["s", j - 1] = store(j - 1)
    if j + 2 < NK:
      d["i", j + 2] = iload(j + 2)
  d["g", NK - 1].wait()
  d["s", NK - 1] = store(NK - 1)
  d["s", NK - 3].wait()
  d["s", NK - 2].wait()
  d["s", NK - 1].wait()

  for s, t, cmax in tails:
    v, x = vs[s], xs[s]
    c = wid + NW * t

    @pl.when(c < cmax)
    def _(v=v, x=x, c=c):
      pltpu.sync_copy(v.at[pl.ds(c * B, B)], idxb[0])
      pltpu.async_copy(ns_hbm.at[idxb[0]], rowb[0], si[0]).wait()
      pltpu.sync_copy(rowb[0], x.at[pl.ds(c * B, B)])


_GATHER_SCRATCH = ([pltpu.VMEM((B,), jnp.int32)] * 3
                   + [pltpu.VMEM((B, H), jnp.float32)] * 3
                   + [pltpu.SemaphoreType.DMA] * 9)


@functools.cache
def _build_gather2():
  steps, tails = _make_steps((CR, CR))

  @functools.partial(
      pl.kernel,
      out_type=(jax.ShapeDtypeStruct((CR * B, H), jnp.float32),
                jax.ShapeDtypeStruct((CR * B, H), jnp.float32)),
      mesh=_mesh(), scratch_types=list(_GATHER_SCRATCH),
  )
  def k(ns_hbm, v0, v1, x0, x1, ib0, ib1, ib2, rb0, rb1, rb2, *sems):
    wid = lax.axis_index("s") * NC + lax.axis_index("c")
    _gather_pipeline(ns_hbm, (v0, v1), (x0, x1), (ib0, ib1, ib2),
                     (rb0, rb1, rb2), sems[0:3], sems[3:6], sems[6:9],
                     wid, steps, tails)

  return k


@functools.cache
def _build_gather3():
  steps, tails = _make_steps((CR, CR, C2))

  @functools.partial(
      pl.kernel,
      out_type=(jax.ShapeDtypeStruct((CR * B, H), jnp.float32),
                jax.ShapeDtypeStruct((CR * B, H), jnp.float32),
                jax.ShapeDtypeStruct((C2 * B, H), jnp.float32)),
      mesh=_mesh(), scratch_types=list(_GATHER_SCRATCH),
  )
  def k(ns_hbm, v0, v1, v2, x0, x1, x2,
        ib0, ib1, ib2, rb0, rb1, rb2, *sems):
    wid = lax.axis_index("s") * NC + lax.axis_index("c")
    _gather_pipeline(ns_hbm, (v0, v1, v2), (x0, x1, x2), (ib0, ib1, ib2),
                     (rb0, rb1, rb2), sems[0:3], sems[3:6], sems[6:9],
                     wid, steps, tails)

  return k


def _scatter_pipeline(es, vs, acc, out_hbm, idxb, bufb, si, sl, sa,
                      cid, sid, wid, steps, tails):
  def zrow(r, carry):
    for j in range(H // 16):
      bufb[0][r, pl.ds(j * 16, 16)] = jnp.zeros((16,), jnp.float32)
    return carry

  lax.fori_loop(0, ZB, zrow, 0)

  def zchunk(k, carry):
    c = sid + NS * k

    @pl.when(c < CZ)
    def _():
      pltpu.sync_copy(bufb[0], acc.at[pl.ds(c * ZB, ZB)])
    return carry

  lax.fori_loop(0, KZ, zchunk, 0)
  plsc.subcore_barrier()

  def srcs(j):
    s, t = steps[j]
    return es[s], vs[s], wid + NW * t

  def iload(j):
    _, v, c = srcs(j)
    m = j % 3
    return pltpu.async_copy(v.at[pl.ds(c * B, B)], idxb[m], si[m])

  def dload(j):
    e, _, c = srcs(j)
    m = j % 3
    return pltpu.async_copy(e.at[pl.ds(c * B, B)], bufb[m], sl[m])

  def addc(j):
    m = j % 3
    return pltpu.async_copy(bufb[m], acc.at[idxb[m]], sa[m], add=True)

  NK = len(steps)
  d = {}
  d["i", 0] = iload(0)
  d["l", 0] = dload(0)
  d["i", 1] = iload(1)
  d["l", 1] = dload(1)
  for j in range(NK):
    d["i", j].wait()
    d["l", j].wait()
    d["a", j] = addc(j)
    if j >= 1:
      d["a", j - 1].wait()
    if j + 2 < NK:
      d["i", j + 2] = iload(j + 2)
      d["l", j + 2] = dload(j + 2)
  d["a", NK - 1].wait()

  for s, t, cmax in tails:
    e, v = es[s], vs[s]
    c = wid + NW * t

    @pl.when(c < cmax)
    def _(e=e, v=v, c=c):
      pltpu.sync_copy(v.at[pl.ds(c * B, B)], idxb[0])
      pltpu.sync_copy(e.at[pl.ds(c * B, B)], bufb[0])
      pltpu.sync_copy(bufb[0], acc.at[idxb[0]], add=True)

  plsc.subcore_barrier()

  def ochunk(k, carry):
    c = sid + NS * k

    @pl.when(c < CZ)
    def _():
      pltpu.sync_copy(acc.at[pl.ds(c * ZB, ZB)],
                      out_hbm.at[cid, pl.ds(c * ZB, ZB)])
    return carry

  lax.fori_loop(0, KZ, ochunk, 0)


_SCATTER_SCRATCH = ([pltpu.VMEM((B,), jnp.int32)] * 3
                    + [pltpu.VMEM((B, H), jnp.float32)] * 3
                    + [pltpu.VMEM_SHARED((N, H), jnp.float32)]
                    + [pltpu.SemaphoreType.DMA] * 9)


@functools.cache
def _build_scatter2():
  steps, tails = _make_steps((CR, CR))

  @functools.partial(
      pl.kernel,
      out_type=jax.ShapeDtypeStruct((NC, N, H), jnp.float32),
      mesh=_mesh(), scratch_types=list(_SCATTER_SCRATCH),
  )
  def k(e0, e1, v0, v1, out_hbm, ib0, ib1, ib2, bb0, bb1, bb2, acc, *sems):
    cid = lax.axis_index("c")
    sid = lax.axis_index("s")
    wid = sid * NC + cid
    _scatter_pipeline((e0, e1), (v0, v1), acc, out_hbm, (ib0, ib1, ib2),
                      (bb0, bb1, bb2), sems[0:3], sems[3:6], sems[6:9],
                      cid, sid, wid, steps, tails)

  return k


@functools.cache
def _build_scatter3():
  steps, tails = _make_steps((CR, CR, C2))

  @functools.partial(
      pl.kernel,
      out_type=jax.ShapeDtypeStruct((NC, N, H), jnp.float32),
      mesh=_mesh(), scratch_types=list(_SCATTER_SCRATCH),
  )
  def k(e0, e1, e2, v0, v1, v2, out_hbm,
        ib0, ib1, ib2, bb0, bb1, bb2, acc, *sems):
    cid = lax.axis_index("c")
    sid = lax.axis_index("s")
    wid = sid * NC + cid
    _scatter_pipeline((e0, e1, e2), (v0, v1, v2), acc, out_hbm,
                      (ib0, ib1, ib2), (bb0, bb1, bb2),
                      sems[0:3], sems[3:6], sems[6:9],
                      cid, sid, wid, steps, tails)

  return k


def _prep_body(has_extra, ns_ref, wgeT_ref, wex0_ref, wexCT_ref, b1_ref,
               c0_ref):
    x = ns_ref[...]
    off = jnp.max(x, axis=0, keepdims=True)
    s = jnp.sum(jnp.exp((x - off) * 8.0), axis=0, keepdims=True)
    ge = 0.125 * jnp.log(s) + off
    c0 = jnp.dot(ge, wgeT_ref[...], preferred_element_type=jnp.float32)
    c0 = c0 + b1_ref[...]
    if has_extra:
        counts = jnp.sum(x[:, :T], axis=0, keepdims=True)
        c0 = c0 + (N / float(MAXOBJ)) * wex0_ref[...]
        c0 = c0 + jnp.dot(counts * (1.0 / N), wexCT_ref[...],
                          preferred_element_type=jnp.float32)
    c0_ref[...] = c0


def _prep(ns, wgeT, wex0, wexCT, b1, has_extra):
    return pl.pallas_call(
        functools.partial(_prep_body, has_extra),
        out_shape=jax.ShapeDtypeStruct((1, 2 * H), jnp.float32),
    )(ns, wgeT, wex0, wexCT, b1)


def _mlpR_body(xa_ref, xb_ref, w1T_ref, b1_ref, w2T_ref, b2_ref,
               ea_ref, eb_ref):
    x16 = jnp.concatenate([xa_ref[...], xb_ref[...]],
                          axis=1).astype(jnp.bfloat16)
    h = jnp.maximum(
        jnp.dot(x16, w1T_ref[...], preferred_element_type=jnp.float32)
        + b1_ref[...], 0.0)
    o = jnp.dot(h.astype(jnp.bfloat16), w2T_ref[...],
                preferred_element_type=jnp.float32) + b2_ref[...]
    e = jnp.exp(8.0 * o)
    ea_ref[...] = e[:, :H]
    eb_ref[...] = e[:, H:]


def _mlpR(xa, xb, w1T, b1, w2T, b2, bf):
    nb = 80000 // bf
    d = 2 * H
    espec = jax.ShapeDtypeStruct((80000, H), jnp.float32)
    return pl.pallas_call(
        _mlpR_body,
        grid=(nb,),
        in_specs=[pl.BlockSpec((bf, H), lambda i: (i, 0)),
                  pl.BlockSpec((bf, H), lambda i: (i, 0)),
                  pl.BlockSpec((d, d), lambda i: (0, 0)),
                  pl.BlockSpec((1, d), lambda i: (0, 0)),
                  pl.BlockSpec((d, d), lambda i: (0, 0)),
                  pl.BlockSpec((1, d), lambda i: (0, 0))],
        out_specs=[pl.BlockSpec((bf, H), lambda i: (i, 0)),
                   pl.BlockSpec((bf, H), lambda i: (i, 0))],
        out_shape=(espec, espec),
    )(xa, xb, w1T, b1, w2T, b2)


def _mlp2_body(x_ref, w1T_ref, b1_ref, w2T_ref, b2_ref, o_ref):
    x16 = x_ref[...].astype(jnp.bfloat16)
    h = jnp.maximum(
        jnp.dot(x16, w1T_ref[...], preferred_element_type=jnp.float32)
        + b1_ref[...], 0.0)
    o = jnp.dot(h.astype(jnp.bfloat16), w2T_ref[...],
                preferred_element_type=jnp.float32)
    o_ref[...] = jnp.exp(8.0 * (o + b2_ref[...]))


def _mlp2(x2, w1T, b1, w2T, b2, bf):
    nb = 10000 // bf
    return pl.pallas_call(
        _mlp2_body,
        grid=(nb,),
        in_specs=[pl.BlockSpec((bf, H), lambda i: (i, 0)),
                  pl.BlockSpec((H, H), lambda i: (0, 0)),
                  pl.BlockSpec((1, H), lambda i: (0, 0)),
                  pl.BlockSpec((H, H), lambda i: (0, 0)),
                  pl.BlockSpec((1, H), lambda i: (0, 0))],
        out_specs=pl.BlockSpec((bf, H), lambda i: (i, 0)),
        out_shape=jax.ShapeDtypeStruct((10000, H), jnp.float32),
    )(x2, w1T, b1, w2T, b2)


def _upd_body(p0_ref, p1_ref, ns_ref, c0_ref, w1mT_ref, w1nsT_ref, w2T_ref,
              b2_ref, o_ref):
    p = p0_ref[0] + p0_ref[1] + p1_ref[0] + p1_ref[1]
    m = 0.125 * jnp.log(p + 1e-16)
    h = jnp.maximum(
        jnp.dot(m, w1mT_ref[...], preferred_element_type=jnp.float32)
        + jnp.dot(ns_ref[...], w1nsT_ref[...],
                  preferred_element_type=jnp.float32)
        + c0_ref[...], 0.0)
    o_ref[...] = jnp.dot(h, w2T_ref[...],
                         preferred_element_type=jnp.float32) + b2_ref[...]


def _upd(p0, p1, ns, c0, w1mT, w1nsT, w2T, b2, bf):
    nb = N // bf
    return pl.pallas_call(
        _upd_body,
        grid=(nb,),
        in_specs=[pl.BlockSpec((NC, bf, H), lambda i: (0, i, 0)),
                  pl.BlockSpec((NC, bf, H), lambda i: (0, i, 0)),
                  pl.BlockSpec((bf, H), lambda i: (i, 0)),
                  pl.BlockSpec((1, 2 * H), lambda i: (0, 0)),
                  pl.BlockSpec((H, 2 * H), lambda i: (0, 0)),
                  pl.BlockSpec((H, 2 * H), lambda i: (0, 0)),
                  pl.BlockSpec((2 * H, H), lambda i: (0, 0)),
                  pl.BlockSpec((1, H), lambda i: (0, 0))],
        out_specs=pl.BlockSpec((bf, H), lambda i: (i, 0)),
        out_shape=jax.ShapeDtypeStruct((N, H), jnp.float32),
    )(p0, p1, ns, c0, w1mT, w1nsT, w2T, b2)


def _gather2_fn(ns, v0, v1):
    return _build_gather2()(ns, v0, v1)


def _gather3_fn(ns, v0, v1, v2):
    return _build_gather3()(ns, v0, v1, v2)


def _scatter2_fn(e0, e1, v0, v1):
    return _build_scatter2()(e0, e1, v0, v1)


def _scatter3_fn(e0, e1, e2, v0, v1, v2):
    return _build_scatter3()(e0, e1, e2, v0, v1, v2)


def kernel(type_ids, rel0_values, rel1_values, rel2_values, init_random,
           r0_W1, r0_b1, r0_W2, r0_b2, r1_W1, r1_b1, r1_W2, r1_b2,
           r2_W1, r2_b1, r2_W2, r2_b2,
           u_W1, u_b1, u_W2, u_b2, v_W1, v_b1, v_W2, v_b2):
    f32 = jnp.float32
    bf16 = jnp.bfloat16
    ns = jnp.concatenate(
        [jax.nn.one_hot(type_ids, T, dtype=f32), init_random], axis=1)
    r0v = rel0_values.astype(jnp.int32)
    r1v = rel1_values.astype(jnp.int32)
    va0, vb0 = r0v[0::2], r0v[1::2]
    va1, vb1 = r1v[0::2], r1v[1::2]
    v2 = rel2_values.astype(jnp.int32)

    w1T0 = r0_W1.T.astype(bf16)
    w2T0 = r0_W2.T.astype(bf16)
    b10 = r0_b1.reshape(1, 2 * H)
    b20 = r0_b2.reshape(1, 2 * H)
    w1T1 = r1_W1.T.astype(bf16)
    w2T1 = r1_W2.T.astype(bf16)
    b11 = r1_b1.reshape(1, 2 * H)
    b21 = r1_b2.reshape(1, 2 * H)
    w1T2 = r2_W1.T.astype(bf16)
    w2T2 = r2_W2.T.astype(bf16)
    b12 = r2_b1.reshape(1, H)
    b22 = r2_b2.reshape(1, H)

    E = T + 1
    v_geT = v_W1[:, E:E + H].T
    v_mT = v_W1[:, E + H:E + 2 * H].T
    v_nsT = v_W1[:, E + 2 * H:].T
    v_ex0 = v_W1[:, 0:1].T
    v_exCT = v_W1[:, 1:E].T
    u_geT = u_W1[:, :H].T
    u_mT = u_W1[:, H:2 * H].T
    u_nsT = u_W1[:, 2 * H:].T
    zpad = jnp.zeros((T, 2 * H), f32)
    zpad1 = jnp.zeros((1, 2 * H), f32)

    for it in range(2):
        if it == 0:
            c0 = _prep(ns, v_geT, v_ex0, v_exCT, v_b1.reshape(1, -1), True)
            w1mT, w1nsT = v_mT, v_nsT
            w2T, b2 = v_W2.T, v_b2.reshape(1, H)
        else:
            c0 = _prep(ns, u_geT, zpad1, zpad, u_b1.reshape(1, -1), False)
            w1mT, w1nsT = u_mT, u_nsT
            w2T, b2 = u_W2.T, u_b2.reshape(1, H)

        xa0, xb0 = _gather2_fn(ns, va0, vb0)
        xa1, xb1, x2 = _gather3_fn(ns, va1, vb1, v2)
        ea0, eb0 = _mlpR(xa0, xb0, w1T0, b10, w2T0, b20, 1000)
        ea1, eb1 = _mlpR(xa1, xb1, w1T1, b11, w2T1, b21, 1000)
        e2 = _mlp2(x2, w1T2, b12, w2T2, b22, 1000)
        p0 = _scatter2_fn(ea0, eb0, va0, vb0)
        p1 = _scatter3_fn(ea1, eb1, e2, va1, vb1, v2)
        ns = _upd(p0, p1, ns, c0, w1mT, w1nsT, w2T, b2, 1000)
    return ns


# speedup vs baseline: 1.1000x; 1.0099x over previous
import functools

import jax
import jax.numpy as jnp
from jax import lax
from jax.experimental import pallas as pl
from jax.experimental.pallas import tpu as pltpu
from jax.experimental.pallas import tpu_sc as plsc

N = 10000
T = 8
H = 128
MAXOBJ = 20000

NC = 2
NS = 16
NW = NC * NS

B = 80
CR = 80000 // B
C2 = 10000 // B
ZB = B
CZ = N // ZB
KZ = -(-CZ // NS)


def _make_steps(counts):
  steps = [(s, t) for s, cnt in enumerate(counts) for t in range(cnt // NW)]
  tails = [(s, cnt // NW, cnt) for s, cnt in enumerate(counts) if cnt % NW]
  return steps, tails


BG = 128
CRG = 80000 // BG


def _mesh():
  return plsc.VectorSubcoreMesh(core_axis_name="c", subcore_axis_name="s",
                                num_cores=NC, num_subcores=NS)


def _gather_pipeline(ns_hbm, vs, xs, bs, idxb, rowb, si, sg, ss, wid, steps,
                     tails):
  def src_dst(j):
    s, t = steps[j]
    return vs[s], xs[s], bs[s], wid + NW * t

  def ibuf(m, b):
    return idxb[m] if b == BG else idxb[m].at[pl.ds(0, b)]

  def rbuf(m, b):
    return rowb[m] if b == BG else rowb[m].at[pl.ds(0, b)]

  def iload(j):
    v, _, b, c = src_dst(j)
    return pltpu.async_copy(v.at[pl.ds(c * b, b)], ibuf(j % 3, b), si[j % 3])

  def gath(j):
    _, _, b, _ = src_dst(j)
    m = j % 3
    return pltpu.async_copy(ns_hbm.at[ibuf(m, b)], rbuf(m, b), sg[m])

  def store(j):
    _, x, b, c = src_dst(j)
    m = j % 3
    return pltpu.async_copy(rbuf(m, b), x.at[pl.ds(c * b, b)], ss[m])

  NK = len(steps)
  d = {}
  d["i", 0] = iload(0)
  d["i", 1] = iload(1)
  for j in range(NK):
    d["i", j].wait()
    if j >= 3:
      d["s", j - 3].wait()
    d["g", j] = gath(j)
    if j >= 1:
      d["g", j - 1].wait()
      d["s", j - 1] = store(j - 1)
    if j + 2 < NK:
      d["i", j + 2] = iload(j + 2)
  d["g", NK - 1].wait()
  d["s", NK - 1] = store(NK - 1)
  d["s", NK - 3].wait()
  d["s", NK - 2].wait()
  d["s", NK - 1].wait()

  for s, t, cmax in tails:
    v, x, b = vs[s], xs[s], bs[s]
    c = wid + NW * t

    @pl.when(c < cmax)
    def _(v=v, x=x, b=b, c=c):
      pltpu.sync_copy(v.at[pl.ds(c * b, b)], ibuf(0, b))
      pltpu.async_copy(ns_hbm.at[ibuf(0, b)], rbuf(0, b), si[0]).wait()
      pltpu.sync_copy(rbuf(0, b), x.at[pl.ds(c * b, b)])


_GATHER_SCRATCH = ([pltpu.VMEM((BG,), jnp.int32)] * 3
                   + [pltpu.VMEM((BG, H), jnp.float32)] * 3
                   + [pltpu.SemaphoreType.DMA] * 9)


@functools.cache
def _build_gather2():
  steps, tails = _make_steps((CRG, CRG))

  @functools.partial(
      pl.kernel,
      out_type=(jax.ShapeDtypeStruct((CRG * BG, H), jnp.float32),
                jax.ShapeDtypeStruct((CRG * BG, H), jnp.float32)),
      mesh=_mesh(), scratch_types=list(_GATHER_SCRATCH),
  )
  def k(ns_hbm, v0, v1, x0, x1, ib0, ib1, ib2, rb0, rb1, rb2, *sems):
    wid = lax.axis_index("s") * NC + lax.axis_index("c")
    _gather_pipeline(ns_hbm, (v0, v1), (x0, x1), (BG, BG), (ib0, ib1, ib2),
                     (rb0, rb1, rb2), sems[0:3], sems[3:6], sems[6:9],
                     wid, steps, tails)

  return k


@functools.cache
def _build_gather3():
  steps, tails = _make_steps((CRG, CRG, C2))

  @functools.partial(
      pl.kernel,
      out_type=(jax.ShapeDtypeStruct((CRG * BG, H), jnp.float32),
                jax.ShapeDtypeStruct((CRG * BG, H), jnp.float32),
                jax.ShapeDtypeStruct((C2 * B, H), jnp.float32)),
      mesh=_mesh(), scratch_types=list(_GATHER_SCRATCH),
  )
  def k(ns_hbm, v0, v1, v2, x0, x1, x2,
        ib0, ib1, ib2, rb0, rb1, rb2, *sems):
    wid = lax.axis_index("s") * NC + lax.axis_index("c")
    _gather_pipeline(ns_hbm, (v0, v1, v2), (x0, x1, x2), (BG, BG, B),
                     (ib0, ib1, ib2), (rb0, rb1, rb2),
                     sems[0:3], sems[3:6], sems[6:9],
                     wid, steps, tails)

  return k


def _scatter_pipeline(es, vs, acc, out_hbm, idxb, bufb, si, sl, sa,
                      cid, sid, wid, steps, tails):
  def zrow(r, carry):
    for j in range(H // 16):
      bufb[0][r, pl.ds(j * 16, 16)] = jnp.zeros((16,), jnp.float32)
    return carry

  lax.fori_loop(0, ZB, zrow, 0)

  def zchunk(k, carry):
    c = sid + NS * k

    @pl.when(c < CZ)
    def _():
      pltpu.sync_copy(bufb[0], acc.at[pl.ds(c * ZB, ZB)])
    return carry

  lax.fori_loop(0, KZ, zchunk, 0)
  plsc.subcore_barrier()

  def srcs(j):
    s, t = steps[j]
    return es[s], vs[s], wid + NW * t

  def iload(j):
    _, v, c = srcs(j)
    m = j % 3
    return pltpu.async_copy(v.at[pl.ds(c * B, B)], idxb[m], si[m])

  def dload(j):
    e, _, c = srcs(j)
    m = j % 3
    return pltpu.async_copy(e.at[pl.ds(c * B, B)], bufb[m], sl[m])

  def addc(j):
    m = j % 3
    return pltpu.async_copy(bufb[m], acc.at[idxb[m]], sa[m], add=True)

  NK = len(steps)
  d = {}
  d["i", 0] = iload(0)
  d["l", 0] = dload(0)
  d["i", 1] = iload(1)
  d["l", 1] = dload(1)
  for j in range(NK):
    d["i", j].wait()
    d["l", j].wait()
    d["a", j] = addc(j)
    if j >= 1:
      d["a", j - 1].wait()
    if j + 2 < NK:
      d["i", j + 2] = iload(j + 2)
      d["l", j + 2] = dload(j + 2)
  d["a", NK - 1].wait()

  for s, t, cmax in tails:
    e, v = es[s], vs[s]
    c = wid + NW * t

    @pl.when(c < cmax)
    def _(e=e, v=v, c=c):
      pltpu.sync_copy(v.at[pl.ds(c * B, B)], idxb[0])
      pltpu.sync_copy(e.at[pl.ds(c * B, B)], bufb[0])
      pltpu.sync_copy(bufb[0], acc.at[idxb[0]], add=True)

  plsc.subcore_barrier()

  def ochunk(k, carry):
    c = sid + NS * k

    @pl.when(c < CZ)
    def _():
      pltpu.sync_copy(acc.at[pl.ds(c * ZB, ZB)],
                      out_hbm.at[cid, pl.ds(c * ZB, ZB)])
    return carry

  lax.fori_loop(0, KZ, ochunk, 0)


_SCATTER_SCRATCH = ([pltpu.VMEM((B,), jnp.int32)] * 3
                    + [pltpu.VMEM((B, H), jnp.float32)] * 3
                    + [pltpu.VMEM_SHARED((N, H), jnp.float32)]
                    + [pltpu.SemaphoreType.DMA] * 9)


@functools.cache
def _build_scatter2():
  steps, tails = _make_steps((CR, CR))

  @functools.partial(
      pl.kernel,
      out_type=jax.ShapeDtypeStruct((NC, N, H), jnp.float32),
      mesh=_mesh(), scratch_types=list(_SCATTER_SCRATCH),
  )
  def k(e0, e1, v0, v1, out_hbm, ib0, ib1, ib2, bb0, bb1, bb2, acc, *sems):
    cid = lax.axis_index("c")
    sid = lax.axis_index("s")
    wid = sid * NC + cid
    _scatter_pipeline((e0, e1), (v0, v1), acc, out_hbm, (ib0, ib1, ib2),
                      (bb0, bb1, bb2), sems[0:3], sems[3:6], sems[6:9],
                      cid, sid, wid, steps, tails)

  return k


@functools.cache
def _build_scatter3():
  steps, tails = _make_steps((CR, CR, C2))

  @functools.partial(
      pl.kernel,
      out_type=jax.ShapeDtypeStruct((NC, N, H), jnp.float32),
      mesh=_mesh(), scratch_types=list(_SCATTER_SCRATCH),
  )
  def k(e0, e1, e2, v0, v1, v2, out_hbm,
        ib0, ib1, ib2, bb0, bb1, bb2, acc, *sems):
    cid = lax.axis_index("c")
    sid = lax.axis_index("s")
    wid = sid * NC + cid
    _scatter_pipeline((e0, e1, e2), (v0, v1, v2), acc, out_hbm,
                      (ib0, ib1, ib2), (bb0, bb1, bb2),
                      sems[0:3], sems[3:6], sems[6:9],
                      cid, sid, wid, steps, tails)

  return k


def _prep_body(has_extra, ns_ref, wgeT_ref, wex0_ref, wexCT_ref, b1_ref,
               c0_ref):
    x = ns_ref[...]
    off = jnp.max(x, axis=0, keepdims=True)
    s = jnp.sum(jnp.exp((x - off) * 8.0), axis=0, keepdims=True)
    ge = 0.125 * jnp.log(s) + off
    c0 = jnp.dot(ge, wgeT_ref[...], preferred_element_type=jnp.float32)
    c0 = c0 + b1_ref[...]
    if has_extra:
        counts = jnp.sum(x[:, :T], axis=0, keepdims=True)
        c0 = c0 + (N / float(MAXOBJ)) * wex0_ref[...]
        c0 = c0 + jnp.dot(counts * (1.0 / N), wexCT_ref[...],
                          preferred_element_type=jnp.float32)
    c0_ref[...] = c0


def _prep(ns, wgeT, wex0, wexCT, b1, has_extra):
    return pl.pallas_call(
        functools.partial(_prep_body, has_extra),
        out_shape=jax.ShapeDtypeStruct((1, 2 * H), jnp.float32),
    )(ns, wgeT, wex0, wexCT, b1)


def _mlpR_body(xa_ref, xb_ref, w1T_ref, b1_ref, w2T_ref, b2_ref,
               ea_ref, eb_ref):
    x16 = jnp.concatenate([xa_ref[...], xb_ref[...]],
                          axis=1).astype(jnp.bfloat16)
    h = jnp.maximum(
        jnp.dot(x16, w1T_ref[...], preferred_element_type=jnp.float32)
        + b1_ref[...], 0.0)
    o = jnp.dot(h.astype(jnp.bfloat16), w2T_ref[...],
                preferred_element_type=jnp.float32) + b2_ref[...]
    e = jnp.exp(8.0 * o)
    ea_ref[...] = e[:, :H]
    eb_ref[...] = e[:, H:]


def _mlpR(xa, xb, w1T, b1, w2T, b2, bf):
    nb = 80000 // bf
    d = 2 * H
    espec = jax.ShapeDtypeStruct((80000, H), jnp.float32)
    return pl.pallas_call(
        _mlpR_body,
        grid=(nb,),
        in_specs=[pl.BlockSpec((bf, H), lambda i: (i, 0)),
                  pl.BlockSpec((bf, H), lambda i: (i, 0)),
                  pl.BlockSpec((d, d), lambda i: (0, 0)),
                  pl.BlockSpec((1, d), lambda i: (0, 0)),
                  pl.BlockSpec((d, d), lambda i: (0, 0)),
                  pl.BlockSpec((1, d), lambda i: (0, 0))],
        out_specs=[pl.BlockSpec((bf, H), lambda i: (i, 0)),
                   pl.BlockSpec((bf, H), lambda i: (i, 0))],
        out_shape=(espec, espec),
    )(xa, xb, w1T, b1, w2T, b2)


def _mlp2_body(x_ref, w1T_ref, b1_ref, w2T_ref, b2_ref, o_ref):
    x16 = x_ref[...].astype(jnp.bfloat16)
    h = jnp.maximum(
        jnp.dot(x16, w1T_ref[...], preferred_element_type=jnp.float32)
        + b1_ref[...], 0.0)
    o = jnp.dot(h.astype(jnp.bfloat16), w2T_ref[...],
                preferred_element_type=jnp.float32)
    o_ref[...] = jnp.exp(8.0 * (o + b2_ref[...]))


def _mlp2(x2, w1T, b1, w2T, b2, bf):
    nb = 10000 // bf
    return pl.pallas_call(
        _mlp2_body,
        grid=(nb,),
        in_specs=[pl.BlockSpec((bf, H), lambda i: (i, 0)),
                  pl.BlockSpec((H, H), lambda i: (0, 0)),
                  pl.BlockSpec((1, H), lambda i: (0, 0)),
                  pl.BlockSpec((H, H), lambda i: (0, 0)),
                  pl.BlockSpec((1, H), lambda i: (0, 0))],
        out_specs=pl.BlockSpec((bf, H), lambda i: (i, 0)),
        out_shape=jax.ShapeDtypeStruct((10000, H), jnp.float32),
    )(x2, w1T, b1, w2T, b2)


def _upd_body(p0_ref, p1_ref, ns_ref, c0_ref, w1mT_ref, w1nsT_ref, w2T_ref,
              b2_ref, o_ref):
    p = p0_ref[0] + p0_ref[1] + p1_ref[0] + p1_ref[1]
    m = 0.125 * jnp.log(p + 1e-16)
    h = jnp.maximum(
        jnp.dot(m, w1mT_ref[...], preferred_element_type=jnp.float32)
        + jnp.dot(ns_ref[...], w1nsT_ref[...],
                  preferred_element_type=jnp.float32)
        + c0_ref[...], 0.0)
    o_ref[...] = jnp.dot(h, w2T_ref[...],
                         preferred_element_type=jnp.float32) + b2_ref[...]


def _upd(p0, p1, ns, c0, w1mT, w1nsT, w2T, b2, bf):
    nb = N // bf
    return pl.pallas_call(
        _upd_body,
        grid=(nb,),
        in_specs=[pl.BlockSpec((NC, bf, H), lambda i: (0, i, 0)),
                  pl.BlockSpec((NC, bf, H), lambda i: (0, i, 0)),
                  pl.BlockSpec((bf, H), lambda i: (i, 0)),
                  pl.BlockSpec((1, 2 * H), lambda i: (0, 0)),
                  pl.BlockSpec((H, 2 * H), lambda i: (0, 0)),
                  pl.BlockSpec((H, 2 * H), lambda i: (0, 0)),
                  pl.BlockSpec((2 * H, H), lambda i: (0, 0)),
                  pl.BlockSpec((1, H), lambda i: (0, 0))],
        out_specs=pl.BlockSpec((bf, H), lambda i: (i, 0)),
        out_shape=jax.ShapeDtypeStruct((N, H), jnp.float32),
    )(p0, p1, ns, c0, w1mT, w1nsT, w2T, b2)


def _gather2_fn(ns, v0, v1):
    return _build_gather2()(ns, v0, v1)


def _gather3_fn(ns, v0, v1, v2):
    return _build_gather3()(ns, v0, v1, v2)


def _scatter2_fn(e0, e1, v0, v1):
    return _build_scatter2()(e0, e1, v0, v1)


def _scatter3_fn(e0, e1, e2, v0, v1, v2):
    return _build_scatter3()(e0, e1, e2, v0, v1, v2)


def kernel(type_ids, rel0_values, rel1_values, rel2_values, init_random,
           r0_W1, r0_b1, r0_W2, r0_b2, r1_W1, r1_b1, r1_W2, r1_b2,
           r2_W1, r2_b1, r2_W2, r2_b2,
           u_W1, u_b1, u_W2, u_b2, v_W1, v_b1, v_W2, v_b2):
    f32 = jnp.float32
    bf16 = jnp.bfloat16
    ns = jnp.concatenate(
        [jax.nn.one_hot(type_ids, T, dtype=f32), init_random], axis=1)
    r0v = rel0_values.astype(jnp.int32)
    r1v = rel1_values.astype(jnp.int32)
    va0, vb0 = r0v[0::2], r0v[1::2]
    va1, vb1 = r1v[0::2], r1v[1::2]
    v2 = rel2_values.astype(jnp.int32)

    w1T0 = r0_W1.T.astype(bf16)
    w2T0 = r0_W2.T.astype(bf16)
    b10 = r0_b1.reshape(1, 2 * H)
    b20 = r0_b2.reshape(1, 2 * H)
    w1T1 = r1_W1.T.astype(bf16)
    w2T1 = r1_W2.T.astype(bf16)
    b11 = r1_b1.reshape(1, 2 * H)
    b21 = r1_b2.reshape(1, 2 * H)
    w1T2 = r2_W1.T.astype(bf16)
    w2T2 = r2_W2.T.astype(bf16)
    b12 = r2_b1.reshape(1, H)
    b22 = r2_b2.reshape(1, H)

    E = T + 1
    v_geT = v_W1[:, E:E + H].T
    v_mT = v_W1[:, E + H:E + 2 * H].T
    v_nsT = v_W1[:, E + 2 * H:].T
    v_ex0 = v_W1[:, 0:1].T
    v_exCT = v_W1[:, 1:E].T
    u_geT = u_W1[:, :H].T
    u_mT = u_W1[:, H:2 * H].T
    u_nsT = u_W1[:, 2 * H:].T
    zpad = jnp.zeros((T, 2 * H), f32)
    zpad1 = jnp.zeros((1, 2 * H), f32)

    for it in range(2):
        if it == 0:
            c0 = _prep(ns, v_geT, v_ex0, v_exCT, v_b1.reshape(1, -1), True)
            w1mT, w1nsT = v_mT, v_nsT
            w2T, b2 = v_W2.T, v_b2.reshape(1, H)
        else:
            c0 = _prep(ns, u_geT, zpad1, zpad, u_b1.reshape(1, -1), False)
            w1mT, w1nsT = u_mT, u_nsT
            w2T, b2 = u_W2.T, u_b2.reshape(1, H)

        xa0, xb0 = _gather2_fn(ns, va0, vb0)
        xa1, xb1, x2 = _gather3_fn(ns, va1, vb1, v2)
        ea0, eb0 = _mlpR(xa0, xb0, w1T0, b10, w2T0, b20, 1000)
        ea1, eb1 = _mlpR(xa1, xb1, w1T1, b11, w2T1, b21, 1000)
        e2 = _mlp2(x2, w1T2, b12, w2T2, b22, 1000)
        p0 = _scatter2_fn(ea0, eb0, va0, vb0)
        p1 = _scatter3_fn(ea1, eb1, e2, va1, vb1, v2)
        ns = _upd(p0, p1, ns, c0, w1mT, w1nsT, w2T, b2, 1000)
    return ns
